# Initial kernel scaffold; baseline (speedup 1.0000x reference)
import jax
import jax.numpy as jnp
from jax.experimental import pallas as pl


def kernel(x, batch, W, b):
    raise NotImplementedError("write your pallas kernel here")


# speedup vs baseline: 12.3422x; 12.3422x over previous
import functools

import jax
import jax.numpy as jnp
from jax.experimental import pallas as pl
from jax.experimental.pallas import tpu as pltpu

_NEG_INF = float("-inf")


def _att_pool_kernel(x_ref, bc_ref, br_ref, w_ref, b_ref, out_ref,
                     m_ref, s_ref, acc_ref, *, nblocks, B):
    k = pl.program_id(0)
    R = x_ref.shape[0]

    @pl.when(k == 0)
    def _init():
        m_ref[...] = jnp.full_like(m_ref, _NEG_INF)
        s_ref[...] = jnp.zeros_like(s_ref)
        acc_ref[...] = jnp.zeros_like(acc_ref)

    xb = x_ref[...]
    bi_col = bc_ref[...]
    bi_row = br_ref[0]

    oh_rb = bi_col == jax.lax.broadcasted_iota(jnp.int32, (R, B), 1)
    oh_br = bi_row == jax.lax.broadcasted_iota(jnp.int32, (B, R), 0)

    g = jnp.dot(xb, w_ref[...], preferred_element_type=jnp.float32)
    g = g + b_ref[...]

    bmax = jnp.max(jnp.where(oh_rb, g, _NEG_INF), axis=0, keepdims=True)
    m_old = m_ref[...]
    m_new = jnp.maximum(m_old, bmax.T)
    m_ref[...] = m_new
    scale = jnp.where(m_old == _NEG_INF, 0.0, jnp.exp(m_old - m_new))

    m_row = jnp.sum(jnp.where(oh_rb, m_new.T, 0.0), axis=1, keepdims=True)
    p = jnp.exp(g - m_row)

    oh_f = oh_rb.astype(jnp.float32)
    s_upd = jnp.sum(oh_f * p, axis=0, keepdims=True)
    s_ref[...] = s_ref[...] * scale + s_upd.T

    pw = p * xb
    upd = jax.lax.dot_general(
        oh_br.astype(jnp.float32), pw,
        (((1,), (0,)), ((), ())),
        preferred_element_type=jnp.float32)
    acc_ref[...] = acc_ref[...] * scale + upd

    @pl.when(k == nblocks - 1)
    def _finish():
        s = s_ref[...]
        out_ref[...] = jnp.where(s > 0.0, acc_ref[...] / s, 0.0)


@functools.partial(jax.jit, static_argnames=("block_rows",))
def _att_pool(x, batch_i32, W, b2, *, block_rows):
    N, D = x.shape
    B = 128
    R = block_rows
    npad = (-N) % R
    if npad:
        x = jnp.concatenate([x, jnp.zeros((npad, D), x.dtype)], axis=0)
        batch_i32 = jnp.concatenate(
            [batch_i32, jnp.full((npad,), B, jnp.int32)], axis=0)
    nb = (N + npad) // R
    bc = batch_i32.reshape(-1, 1)
    br = batch_i32.reshape(nb, 1, R)

    grid = (nb,)
    kernel_fn = functools.partial(_att_pool_kernel, nblocks=nb, B=B)
    return pl.pallas_call(
        kernel_fn,
        grid=grid,
        in_specs=[
            pl.BlockSpec((R, D), lambda k: (k, 0)),
            pl.BlockSpec((R, 1), lambda k: (k, 0)),
            pl.BlockSpec((1, 1, R), lambda k: (k, 0, 0)),
            pl.BlockSpec((D, 1), lambda k: (0, 0)),
            pl.BlockSpec((1, 1), lambda k: (0, 0)),
        ],
        out_specs=pl.BlockSpec((B, D), lambda k: (0, 0)),
        out_shape=jax.ShapeDtypeStruct((B, D), jnp.float32),
        scratch_shapes=[
            pltpu.VMEM((B, 1), jnp.float32),
            pltpu.VMEM((B, 1), jnp.float32),
            pltpu.VMEM((B, D), jnp.float32),
        ],
        compiler_params=pltpu.CompilerParams(
            dimension_semantics=("arbitrary",),
        ),
    )(x, bc, br, W, b2)


def kernel(x, batch, W, b):
    batch_i32 = batch.astype(jnp.int32)
    b2 = b.reshape(1, 1).astype(jnp.float32)
    return _att_pool(x, batch_i32, W, b2, block_rows=2000)
